# Initial kernel scaffold; baseline (speedup 1.0000x reference)
import jax
import jax.numpy as jnp
from jax.experimental import pallas as pl


def kernel(x, temporal_edge_i, temporal_edge_w, edge_index, edge_weight, Wt, bt, Ws, bs, W_ih, W_hh, b_ih, b_hh, mu_w, mu_b, sigma_w, sigma_b, pi_w, pi_b):
    raise NotImplementedError("write your pallas kernel here")


# speedup vs baseline: 12.8236x; 12.8236x over previous
import jax
import jax.numpy as jnp
from jax.experimental import pallas as pl
from jax.experimental.pallas import tpu as pltpu

B = 32
T = 64
F = 128
H = 256
M = 5
OUT = 128
E_T = 512
E_S = 128

_F32 = jnp.float32


def _dot(a, b):
    return jax.lax.dot_general(a, b, (((1,), (0,)), ((), ())),
                               preferred_element_type=_F32)


def _dot_t(a, b):
    return jax.lax.dot_general(a, b, (((1,), (1,)), ((), ())),
                               preferred_element_type=_F32)


def _dot_lt(a, b):
    return jax.lax.dot_general(a, b, (((0,), (0,)), ((), ())),
                               preferred_element_type=_F32)


def _graph_proj_kernel(x_ref, tei_ref, tew_ref, ei_ref, ew_ref,
                       wt_ref, bt_ref, ws_ref, bs_ref, wih_ref, bg_ref,
                       p_ref, afs_ref, abs_ref):
    b = pl.program_id(0)

    @pl.when(b == 0)
    def _():
        src = ei_ref[0:1, :].astype(jnp.int32)
        dst = ei_ref[1:2, :].astype(jnp.int32)
        w = ew_ref[...]
        iota = jax.lax.broadcasted_iota(jnp.int32, (F, E_S), 0)
        gs = (iota == src).astype(_F32)
        gd = (iota == dst).astype(_F32)
        deg_out = jnp.sum(gs * w, axis=1, keepdims=True)
        deg_in = jnp.sum(gd * w, axis=1, keepdims=True)
        dso = jnp.where(deg_out > 0, deg_out, 1.0)
        dsi = jnp.where(deg_in > 0, deg_in, 1.0)
        w_fwd = w / jnp.sum(gs * dso, axis=0, keepdims=True)
        w_bwd = w / jnp.sum(gd * dsi, axis=0, keepdims=True)
        afs_ref[...] = _dot_t(gs, gd * w_fwd)
        abs_ref[...] = _dot_t(gd, gs * w_bwd)

    src = tei_ref[0, 0:1, :]
    dst = tei_ref[0, 1:2, :]
    w = tew_ref[0]
    iota = jax.lax.broadcasted_iota(jnp.int32, (T, E_T), 0)
    gs = (iota == src).astype(_F32)
    gd = (iota == dst).astype(_F32)
    deg_out = jnp.sum(gs * w, axis=1, keepdims=True)
    deg_in = jnp.sum(gd * w, axis=1, keepdims=True)
    dso = jnp.where(deg_out > 0, deg_out, 1.0)
    dsi = jnp.where(deg_in > 0, deg_in, 1.0)
    w_fwd = w / jnp.sum(gs * dso, axis=0, keepdims=True)
    w_bwd = w / jnp.sum(gd * dsi, axis=0, keepdims=True)
    a_f = _dot_t(gd, gs * w_fwd)
    a_b = _dot_t(gs, gd * w_bwd)

    xb = x_ref[0]
    zf1 = _dot(a_f, xb)
    zf2 = _dot(a_f, zf1)
    zb1 = _dot(a_b, xb)
    zb2 = _dot(a_b, zb1)
    dt = (_dot(zf1, wt_ref[0:F]) + _dot(zf2, wt_ref[F:2 * F])
          + _dot(zb1, wt_ref[2 * F:3 * F]) + _dot(zb2, wt_ref[3 * F:4 * F])
          + bt_ref[...])

    y1 = _dot(xb, afs_ref[...])
    y2 = _dot(y1, afs_ref[...])
    y3 = _dot(xb, abs_ref[...])
    y4 = _dot(y3, abs_ref[...])
    ds = (_dot_lt(ws_ref[0:T], y1) + _dot_lt(ws_ref[T:2 * T], y2)
          + _dot_lt(ws_ref[2 * T:3 * T], y3) + _dot_lt(ws_ref[3 * T:4 * T], y4)
          + bs_ref[...])

    p_ref[0] = (_dot_t(dt, wih_ref[:, 0:H])
                + _dot_t(ds, wih_ref[:, H:H + F])
                + _dot_t(xb, wih_ref[:, H + F:H + 2 * F])
                + bg_ref[...])


def _lstm_head_kernel(p_ref, whh_ref, muw_ref, mub_ref, sgw_ref, sgb_ref,
                      piw_ref, pib_ref, mu_ref, sg_ref, pi_ref, hs_ref):
    def step(t, carry):
        h, c = carry
        gates = p_ref[:, pl.ds(t, 1), :].reshape(B, 4 * H) + _dot_t(h, whh_ref[...])
        i = jax.nn.sigmoid(gates[:, 0:H])
        f = jax.nn.sigmoid(gates[:, H:2 * H])
        g = jnp.tanh(gates[:, 2 * H:3 * H])
        o = jax.nn.sigmoid(gates[:, 3 * H:4 * H])
        c2 = f * c + i * g
        h2 = o * jnp.tanh(c2)
        hs_ref[:, pl.ds(t, 1), :] = h2.reshape(B, 1, H)
        return (h2, c2)

    zeros = jnp.zeros((B, H), _F32)
    jax.lax.fori_loop(0, T, step, (zeros, zeros))

    hs = hs_ref[...].reshape(B * T, H)
    mu_ref[...] = _dot_t(hs, muw_ref[...]) + mub_ref[...]
    sg_ref[...] = jnp.exp(_dot_t(hs, sgw_ref[...]) + sgb_ref[...])
    logits = _dot_t(hs, piw_ref[...]) + pib_ref[...]
    mx = jnp.max(logits, axis=-1, keepdims=True)
    e = jnp.exp(logits - mx)
    pi_ref[...] = e / jnp.sum(e, axis=-1, keepdims=True)


def kernel(x, temporal_edge_i, temporal_edge_w, edge_index, edge_weight,
           Wt, bt, Ws, bs, W_ih, W_hh, b_ih, b_hh,
           mu_w, mu_b, sigma_w, sigma_b, pi_w, pi_b, interpret=False):
    bg = (b_ih + b_hh)[None, :]

    p = pl.pallas_call(
        _graph_proj_kernel,
        grid=(B,),
        in_specs=[
            pl.BlockSpec((1, T, F), lambda b: (b, 0, 0)),
            pl.BlockSpec((1, 2, E_T), lambda b: (b, 0, 0)),
            pl.BlockSpec((1, 1, E_T), lambda b: (b, 0, 0)),
            pl.BlockSpec((2, E_S), lambda b: (0, 0)),
            pl.BlockSpec((1, E_S), lambda b: (0, 0)),
            pl.BlockSpec((4 * F, H), lambda b: (0, 0)),
            pl.BlockSpec((1, H), lambda b: (0, 0)),
            pl.BlockSpec((4 * T, T), lambda b: (0, 0)),
            pl.BlockSpec((T, 1), lambda b: (0, 0)),
            pl.BlockSpec((4 * H, H + 2 * F), lambda b: (0, 0)),
            pl.BlockSpec((1, 4 * H), lambda b: (0, 0)),
        ],
        out_specs=pl.BlockSpec((1, T, 4 * H), lambda b: (b, 0, 0)),
        out_shape=jax.ShapeDtypeStruct((B, T, 4 * H), _F32),
        scratch_shapes=[pltpu.VMEM((F, F), _F32), pltpu.VMEM((F, F), _F32)],
        interpret=interpret,
    )(x, temporal_edge_i, temporal_edge_w[:, None, :], edge_index,
      edge_weight[None, :], Wt, bt[None, :], Ws, bs[:, None], W_ih, bg)

    mu_f, sg_f, pi_f = pl.pallas_call(
        _lstm_head_kernel,
        out_shape=[
            jax.ShapeDtypeStruct((B * T, M * OUT), _F32),
            jax.ShapeDtypeStruct((B * T, M * OUT), _F32),
            jax.ShapeDtypeStruct((B * T, M), _F32),
        ],
        scratch_shapes=[pltpu.VMEM((B, T, H), _F32)],
        interpret=interpret,
    )(p, W_hh, mu_w, mu_b[None, :], sigma_w, sigma_b[None, :],
      pi_w, pi_b[None, :])

    mu = mu_f.reshape(B, T, M, OUT)
    sigma = sg_f.reshape(B, T, M, OUT)
    pi = pi_f.reshape(B, T, M)
    return mu, sigma, pi


# speedup vs baseline: 15.7899x; 1.2313x over previous
import jax
import jax.numpy as jnp
from jax.experimental import pallas as pl
from jax.experimental.pallas import tpu as pltpu

B = 32
T = 64
F = 128
H = 256
M = 5
OUT = 128
E_T = 512
E_S = 128
XIN = H + 2 * F

_F32 = jnp.float32


def _dot(a, b):
    return jax.lax.dot_general(a, b, (((1,), (0,)), ((), ())),
                               preferred_element_type=_F32)


def _dot_t(a, b):
    return jax.lax.dot_general(a, b, (((1,), (1,)), ((), ())),
                               preferred_element_type=_F32)


def _dot_lt(a, b):
    return jax.lax.dot_general(a, b, (((0,), (0,)), ((), ())),
                               preferred_element_type=_F32)


G = 8


def _graph_kernel(x_ref, tei_ref, tew_ref, ei_ref, ew_ref,
                  wt_ref, bt_ref, ws_ref, bs_ref,
                  xin_ref, afs_ref, abs_ref):
    b = pl.program_id(0)

    @pl.when(b == 0)
    def _():
        src = ei_ref[0:1, :].astype(jnp.int32)
        dst = ei_ref[1:2, :].astype(jnp.int32)
        w = ew_ref[...]
        iota = jax.lax.broadcasted_iota(jnp.int32, (F, E_S), 0)
        gs = (iota == src).astype(_F32)
        gd = (iota == dst).astype(_F32)
        deg_out = jnp.sum(gs * w, axis=1, keepdims=True)
        deg_in = jnp.sum(gd * w, axis=1, keepdims=True)
        dso = jnp.where(deg_out > 0, deg_out, 1.0)
        dsi = jnp.where(deg_in > 0, deg_in, 1.0)
        w_fwd = w / jnp.sum(gs * dso, axis=0, keepdims=True)
        w_bwd = w / jnp.sum(gd * dsi, axis=0, keepdims=True)
        afs_ref[...] = _dot_t(gs, gd * w_fwd)
        abs_ref[...] = _dot_t(gd, gs * w_bwd)

    for j in range(G):
        src = tei_ref[j, 0:1, :]
        dst = tei_ref[j, 1:2, :]
        w = tew_ref[j]
        iota = jax.lax.broadcasted_iota(jnp.int32, (T, E_T), 0)
        gs = (iota == src).astype(_F32)
        gd = (iota == dst).astype(_F32)
        deg_out = jnp.sum(gs * w, axis=1, keepdims=True)
        deg_in = jnp.sum(gd * w, axis=1, keepdims=True)
        dso = jnp.where(deg_out > 0, deg_out, 1.0)
        dsi = jnp.where(deg_in > 0, deg_in, 1.0)
        w_fwd = w / jnp.sum(gs * dso, axis=0, keepdims=True)
        w_bwd = w / jnp.sum(gd * dsi, axis=0, keepdims=True)
        a_f = _dot_t(gd, gs * w_fwd)
        a_b = _dot_t(gs, gd * w_bwd)

        xb = x_ref[j]
        zf1 = _dot(a_f, xb)
        zf2 = _dot(a_f, zf1)
        zb1 = _dot(a_b, xb)
        zb2 = _dot(a_b, zb1)
        dt = (_dot(zf1, wt_ref[0:F]) + _dot(zf2, wt_ref[F:2 * F])
              + _dot(zb1, wt_ref[2 * F:3 * F]) + _dot(zb2, wt_ref[3 * F:4 * F])
              + bt_ref[...])

        y1 = _dot(xb, afs_ref[...])
        y2 = _dot(y1, afs_ref[...])
        y3 = _dot(xb, abs_ref[...])
        y4 = _dot(y3, abs_ref[...])
        ds = (_dot_lt(ws_ref[0:T], y1) + _dot_lt(ws_ref[T:2 * T], y2)
              + _dot_lt(ws_ref[2 * T:3 * T], y3)
              + _dot_lt(ws_ref[3 * T:4 * T], y4)
              + bs_ref[...])

        xin_ref[j] = jnp.concatenate([dt, ds, xb], axis=1)


def _proj_lstm_head_kernel(xin_ref, wih_ref, bg_ref, whh_ref,
                           muw_ref, mub_ref, sgw_ref, sgb_ref,
                           piw_ref, pib_ref,
                           mu_ref, sg_ref, pi_ref, p_ref, hs_ref):
    xin = xin_ref[...].reshape(B * T, XIN)
    p_ref[...] = (_dot_t(xin, wih_ref[...]) + bg_ref[...]).reshape(B, T, 4 * H)

    def step(t, carry):
        h, c = carry
        gates = p_ref[:, pl.ds(t, 1), :].reshape(B, 4 * H) + _dot_t(h, whh_ref[...])
        i = jax.nn.sigmoid(gates[:, 0:H])
        f = jax.nn.sigmoid(gates[:, H:2 * H])
        g = jnp.tanh(gates[:, 2 * H:3 * H])
        o = jax.nn.sigmoid(gates[:, 3 * H:4 * H])
        c2 = f * c + i * g
        h2 = o * jnp.tanh(c2)
        hs_ref[:, pl.ds(t, 1), :] = h2.reshape(B, 1, H)
        return (h2, c2)

    zeros = jnp.zeros((B, H), _F32)
    jax.lax.fori_loop(0, T, step, (zeros, zeros))

    hs = hs_ref[...].reshape(B * T, H)
    mu_ref[...] = _dot_t(hs, muw_ref[...]) + mub_ref[...]
    sg_ref[...] = jnp.exp(_dot_t(hs, sgw_ref[...]) + sgb_ref[...])
    logits = _dot_t(hs, piw_ref[...]) + pib_ref[...]
    mx = jnp.max(logits, axis=-1, keepdims=True)
    e = jnp.exp(logits - mx)
    pi_ref[...] = e / jnp.sum(e, axis=-1, keepdims=True)


def kernel(x, temporal_edge_i, temporal_edge_w, edge_index, edge_weight,
           Wt, bt, Ws, bs, W_ih, W_hh, b_ih, b_hh,
           mu_w, mu_b, sigma_w, sigma_b, pi_w, pi_b, interpret=False):
    bg = (b_ih + b_hh)[None, :]

    xin = pl.pallas_call(
        _graph_kernel,
        grid=(B // G,),
        in_specs=[
            pl.BlockSpec((G, T, F), lambda b: (b, 0, 0)),
            pl.BlockSpec((G, 2, E_T), lambda b: (b, 0, 0)),
            pl.BlockSpec((G, 1, E_T), lambda b: (b, 0, 0)),
            pl.BlockSpec((2, E_S), lambda b: (0, 0)),
            pl.BlockSpec((1, E_S), lambda b: (0, 0)),
            pl.BlockSpec((4 * F, H), lambda b: (0, 0)),
            pl.BlockSpec((1, H), lambda b: (0, 0)),
            pl.BlockSpec((4 * T, T), lambda b: (0, 0)),
            pl.BlockSpec((T, 1), lambda b: (0, 0)),
        ],
        out_specs=pl.BlockSpec((G, T, XIN), lambda b: (b, 0, 0)),
        out_shape=jax.ShapeDtypeStruct((B, T, XIN), _F32),
        scratch_shapes=[pltpu.VMEM((F, F), _F32), pltpu.VMEM((F, F), _F32)],
        interpret=interpret,
    )(x, temporal_edge_i, temporal_edge_w[:, None, :], edge_index,
      edge_weight[None, :], Wt, bt[None, :], Ws, bs[:, None])

    mu_f, sg_f, pi_f = pl.pallas_call(
        _proj_lstm_head_kernel,
        out_shape=[
            jax.ShapeDtypeStruct((B * T, M * OUT), _F32),
            jax.ShapeDtypeStruct((B * T, M * OUT), _F32),
            jax.ShapeDtypeStruct((B * T, M), _F32),
        ],
        scratch_shapes=[pltpu.VMEM((B, T, 4 * H), _F32),
                        pltpu.VMEM((B, T, H), _F32)],
        interpret=interpret,
    )(xin, W_ih, bg, W_hh, mu_w, mu_b[None, :], sigma_w, sigma_b[None, :],
      pi_w, pi_b[None, :])

    mu = mu_f.reshape(B, T, M, OUT)
    sigma = sg_f.reshape(B, T, M, OUT)
    pi = pi_f.reshape(B, T, M)
    return mu, sigma, pi
